# baseline (speedup 1.0000x reference)
import jax
import jax.numpy as jnp
from jax.experimental import pallas as pl
from jax.experimental.pallas import tpu as pltpu

_N = 131
_LATDIM = 512
_HYPERNUM = 512
_GNN_LAYER = 2
_CB = 128
_NB = _LATDIM // _CB

_CONTRACT_LANES = (((1,), (1,)), ((), ()))


def _fused_kernel(adj_v, u_v, i_v, uh_v, ih_v,
                  out_b, gnn_b, hyp_b,
                  g_v, emb_v):
    f32 = jnp.float32
    s = pl.program_id(0)

    @pl.when(s == 0)
    def _prep():
        u = u_v[...]
        i = i_v[...]
        emb_v[...] = u + i
        uu = jnp.dot(u, uh_v[...], preferred_element_type=f32)
        ii = jnp.dot(i, ih_v[...], preferred_element_type=f32)
        g_v[...] = (jax.lax.dot_general(uu, uu, _CONTRACT_LANES,
                                        preferred_element_type=f32)
                    + jax.lax.dot_general(ii, ii, _CONTRACT_LANES,
                                          preferred_element_type=f32))

    @pl.when(s > 0)
    def _cols():
        c0 = (s - 1) * _CB
        e_c = emb_v[:, pl.ds(c0, _CB)]
        adj = adj_v[...]
        g = g_v[...]
        tem0 = jnp.dot(adj, e_c, preferred_element_type=f32)
        h0 = jnp.dot(g, e_c, preferred_element_type=f32)
        lat1 = tem0 + h0
        tem1 = jnp.dot(adj, lat1, preferred_element_type=f32)
        h1 = jnp.dot(g, lat1, preferred_element_type=f32)
        gnn_b[0] = tem0
        gnn_b[1] = tem1
        hyp_b[0] = h0
        hyp_b[1] = h1
        out_b[...] = 0.0101 * (e_c + lat1 + (tem1 + h1))


def kernel(adj, uEmbeds, iEmbeds, uHyper, iHyper):
    f32 = jnp.float32
    out_shapes = (
        jax.ShapeDtypeStruct((_N, _LATDIM), f32),
        jax.ShapeDtypeStruct((_GNN_LAYER, _N, _LATDIM), f32),
        jax.ShapeDtypeStruct((_GNN_LAYER, _N, _LATDIM), f32),
    )

    def _cidx(s):
        return jnp.maximum(s - 1, 0)

    return pl.pallas_call(
        _fused_kernel,
        grid=(_NB + 1,),
        in_specs=[
            pl.BlockSpec((_N, _N), lambda s: (0, 0)),
            pl.BlockSpec((_N, _LATDIM), lambda s: (0, 0)),
            pl.BlockSpec((_N, _LATDIM), lambda s: (0, 0)),
            pl.BlockSpec((_LATDIM, _HYPERNUM), lambda s: (0, 0)),
            pl.BlockSpec((_LATDIM, _HYPERNUM), lambda s: (0, 0)),
        ],
        out_specs=(
            pl.BlockSpec((_N, _CB), lambda s: (0, _cidx(s))),
            pl.BlockSpec((_GNN_LAYER, _N, _CB), lambda s: (0, 0, _cidx(s))),
            pl.BlockSpec((_GNN_LAYER, _N, _CB), lambda s: (0, 0, _cidx(s))),
        ),
        out_shape=out_shapes,
        scratch_shapes=[
            pltpu.VMEM((_N, _N), f32),
            pltpu.VMEM((_N, _LATDIM), f32),
        ],
    )(adj, uEmbeds, iEmbeds, uHyper, iHyper)
